# Initial kernel scaffold; baseline (speedup 1.0000x reference)
import jax
import jax.numpy as jnp
from jax.experimental import pallas as pl


def kernel(x, node_attr, mask_downstream_adj, mask_khop_up_adj, full_path_edge_attr_adj, outlet_index, W_dyn, b_dyn, W_film, b_film, W_pos1, b_pos1, w_pos2, b_pos2, W_zr0, b_zr0, W_c0, b_c0, W_zr1, b_zr1, W_c1, b_c1, W_out, b_out):
    raise NotImplementedError("write your pallas kernel here")


# speedup vs baseline: 249.2703x; 249.2703x over previous
import functools

import jax
import jax.numpy as jnp
from jax.experimental import pallas as pl
from jax.experimental.pallas import tpu as pltpu

B = 2
N = 256
T = 24
F = 8
H = 128
FE = 8
POS = 32
O = 32
P = 8
TAILK = 12
BN = B * N

_EW_CHUNK = 8192
_NE = B * N * N


def _ew_kernel(ea_ref, md_ref, mu_ref, w1_ref, b1_ref, w2_ref, b2_ref, o_ref):
    h = jnp.dot(ea_ref[...], w1_ref[...], preferred_element_type=jnp.float32)
    h = jnp.maximum(h + b1_ref[...], 0.0)
    s = jnp.dot(h, w2_ref[...], preferred_element_type=jnp.float32) + b2_ref[...]
    ew = jax.nn.sigmoid(s)
    m = jnp.minimum(md_ref[...] + mu_ref[...], 1.0)
    o_ref[...] = m * ew


def _gru_kernel(a_ref, xtp_ref, na_ref, outlet_ref,
                w_dyn_ref, b_dyn_ref, w_film_ref, b_film_ref,
                wzr0_ref, bzr0_ref, wc0_ref, bc0_ref,
                wzr1_ref, bzr1_ref, wc1_ref, bc1_ref,
                w_out_ref, b_out_ref,
                out_ref, s0_ref, s1_ref):
    f32 = jnp.float32

    film = jnp.dot(na_ref[...], w_film_ref[...], preferred_element_type=f32)
    film = film + b_film_ref[...]
    scale = 1.0 + film[:, :H]
    beta = film[:, H:]
    w_dyn = w_dyn_ref[...]
    b_dyn = b_dyn_ref[...]
    for t in range(T):
        xt = xtp_ref[t * F:(t + 1) * F, :]
        ht = jax.lax.dot_general(xt, w_dyn, (((0,), (0,)), ((), ())),
                                 preferred_element_type=f32)
        s0_ref[t] = jnp.maximum((ht + b_dyn) * scale + beta, 0.0)

    a0 = a_ref[0]
    a1 = a_ref[1]

    def prop(v):
        g0 = jnp.dot(a0, v[:N, :], preferred_element_type=f32)
        g1 = jnp.dot(a1, v[N:, :], preferred_element_type=f32)
        return jnp.concatenate([g0, g1], axis=0)

    acc = jnp.zeros((BN, H), f32)
    for layer in range(2):
        wzr = (wzr0_ref if layer == 0 else wzr1_ref)[...]
        bzr = (bzr0_ref if layer == 0 else bzr1_ref)[...]
        wc = (wc0_ref if layer == 0 else wc1_ref)[...]
        bc = (bc0_ref if layer == 0 else bc1_ref)[...]
        wzr_x, wzr_ax, wzr_h, wzr_ah = (wzr[:H], wzr[H:2 * H],
                                        wzr[2 * H:3 * H], wzr[3 * H:])
        wc_x, wc_ax, wc_h, wc_ah = (wc[:H], wc[H:2 * H],
                                    wc[2 * H:3 * H], wc[3 * H:])
        s_in = s0_ref if layer == 0 else s1_ref
        h = jnp.zeros((BN, H), f32)
        for t in range(T):
            xt = s_in[t]
            agg_x = prop(xt)
            agg_h = prop(h)
            pre_zr = (jnp.dot(xt, wzr_x, preferred_element_type=f32)
                      + jnp.dot(agg_x, wzr_ax, preferred_element_type=f32)
                      + jnp.dot(h, wzr_h, preferred_element_type=f32)
                      + jnp.dot(agg_h, wzr_ah, preferred_element_type=f32)
                      + bzr)
            zr = jax.nn.sigmoid(pre_zr)
            z = zr[:, :H]
            r = zr[:, H:]
            pre_c = (jnp.dot(xt, wc_x, preferred_element_type=f32)
                     + jnp.dot(agg_x, wc_ax, preferred_element_type=f32)
                     + jnp.dot(r * h, wc_h, preferred_element_type=f32)
                     + jnp.dot(r * agg_h, wc_ah, preferred_element_type=f32)
                     + bc)
            c = jnp.tanh(pre_c)
            h = z * h + (1.0 - z) * c
            if layer == 0:
                s1_ref[t] = h
            elif t >= T - TAILK:
                acc = acc + h

    tail = acc * (1.0 / TAILK)
    npred = jnp.dot(tail, w_out_ref[...], preferred_element_type=f32)
    npred = npred + b_out_ref[...]

    iota = jax.lax.broadcasted_iota(jnp.int32, (N, O), 0)
    for b in range(B):
        idx = outlet_ref[b:b + 1, :]
        onehot = (iota == idx).astype(f32)
        nb = npred[b * N:(b + 1) * N, :]
        yb = jax.lax.dot_general(nb, onehot, (((0,), (0,)), ((), ())),
                                 preferred_element_type=f32)
        out_ref[b] = yb


@jax.jit
def kernel(x, node_attr, mask_downstream_adj, mask_khop_up_adj,
           full_path_edge_attr_adj, outlet_index,
           W_dyn, b_dyn, W_film, b_film, W_pos1, b_pos1, w_pos2, b_pos2,
           W_zr0, b_zr0, W_c0, b_c0, W_zr1, b_zr1, W_c1, b_c1,
           W_out, b_out):
    f32 = jnp.float32

    ea2 = full_path_edge_attr_adj.reshape(_NE, FE)
    md2 = mask_downstream_adj.reshape(_NE, 1)
    mu2 = mask_khop_up_adj.reshape(_NE, 1)
    n_chunks = _NE // _EW_CHUNK
    wspec = lambda shape: pl.BlockSpec(shape, lambda i: (0, 0))
    a_flat = pl.pallas_call(
        _ew_kernel,
        grid=(n_chunks,),
        in_specs=[
            pl.BlockSpec((_EW_CHUNK, FE), lambda i: (i, 0)),
            pl.BlockSpec((_EW_CHUNK, 1), lambda i: (i, 0)),
            pl.BlockSpec((_EW_CHUNK, 1), lambda i: (i, 0)),
            wspec((FE, POS)),
            wspec((1, POS)),
            wspec((POS, 1)),
            wspec((1, 1)),
        ],
        out_specs=pl.BlockSpec((_EW_CHUNK, 1), lambda i: (i, 0)),
        out_shape=jax.ShapeDtypeStruct((_NE, 1), f32),
    )(ea2, md2, mu2, W_pos1, b_pos1.reshape(1, POS), w_pos2,
      b_pos2.reshape(1, 1))
    a = a_flat.reshape(B, N, N)

    xtp = jnp.transpose(x, (2, 3, 0, 1)).reshape(T * F, BN)
    na = node_attr.reshape(BN, -1)

    y = pl.pallas_call(
        _gru_kernel,
        out_shape=jax.ShapeDtypeStruct((B, P, O), f32),
        scratch_shapes=[
            pltpu.VMEM((T, BN, H), f32),
            pltpu.VMEM((T, BN, H), f32),
        ],
    )(a, xtp, na, outlet_index,
      W_dyn, b_dyn.reshape(1, H), W_film, b_film.reshape(1, 2 * H),
      W_zr0, b_zr0.reshape(1, 2 * H), W_c0, b_c0.reshape(1, H),
      W_zr1, b_zr1.reshape(1, 2 * H), W_c1, b_c1.reshape(1, H),
      W_out, b_out.reshape(1, P))
    return y
